# Initial kernel scaffold; baseline (speedup 1.0000x reference)
import jax
import jax.numpy as jnp
from jax.experimental import pallas as pl


def kernel(affines):
    raise NotImplementedError("write your pallas kernel here")


# speedup vs baseline: 4.9000x; 4.9000x over previous
import functools

import numpy as np
import jax
import jax.numpy as jnp
from jax import lax
from jax.experimental import pallas as pl
from jax.experimental.pallas import tpu as pltpu
from jax.experimental.pallas import tpu_sc as plsc

_N = 8192
_K = 32
_PED = 64
_HALF = _PED // 2
_RB = 256
_NBLK = _N // _RB
_NW = 32
_BPW = _N * _K // _NW


def _topk_body(pt_ref, pb_ref, idx_ref, d_ref):
    b = pl.program_id(0)
    pt = pt_ref[...]
    pb = pb_ref[...]
    sqc = jnp.sum(pt * pt, axis=0, keepdims=True)
    sqr = jnp.sum(pb * pb, axis=1, keepdims=True)
    dot = lax.dot_general(pb, pt, (((1,), (0,)), ((), ())),
                          preferred_element_type=jnp.float32)
    d = sqr + sqc - 2.0 * dot
    row = lax.broadcasted_iota(jnp.int32, (_RB, _N), 0)
    col = lax.broadcasted_iota(jnp.int32, (_RB, _N), 1)
    d_ref[...] = jnp.where(col == row + b * _RB, jnp.inf, d)

    kio = lax.broadcasted_iota(jnp.int32, (_RB, _K), 1)

    def body(j, out):
        dd = d_ref[...]
        m = jnp.min(dd, axis=1, keepdims=True)
        amin = jnp.min(jnp.where(dd == m, col, _N), axis=1, keepdims=True)
        d_ref[...] = jnp.where(col == amin, jnp.inf, dd)
        return jnp.where(kio == j, amin, out)

    idx_ref[...] = lax.fori_loop(0, _K, body, jnp.zeros((_RB, _K), jnp.int32))


def _topk_call(ppad):
    return pl.pallas_call(
        _topk_body,
        grid=(_NBLK,),
        in_specs=[
            pl.BlockSpec((8, _N), lambda b: (0, 0)),
            pl.BlockSpec((_RB, 8), lambda b: (b, 0)),
        ],
        out_specs=pl.BlockSpec((_RB, _K), lambda b: (b, 0)),
        out_shape=jax.ShapeDtypeStruct((_N, _K), jnp.int32),
        scratch_shapes=[pltpu.VMEM((_RB, _N), jnp.float32)],
    )(ppad.T, ppad)


def _gather_sc(posx, posy, posz, idx_flat):
    mesh = plsc.VectorSubcoreMesh(core_axis_name="c", subcore_axis_name="s")

    @functools.partial(
        pl.kernel,
        mesh=mesh,
        out_type=[
            jax.ShapeDtypeStruct((_N * _K,), jnp.float32),
            jax.ShapeDtypeStruct((_N * _K,), jnp.float32),
            jax.ShapeDtypeStruct((_N * _K,), jnp.float32),
        ],
        scratch_types=[
            pltpu.VMEM((_N,), jnp.float32),
            pltpu.VMEM((_N,), jnp.float32),
            pltpu.VMEM((_N,), jnp.float32),
            pltpu.VMEM((_BPW,), jnp.int32),
            pltpu.VMEM((_BPW,), jnp.float32),
            pltpu.VMEM((_BPW,), jnp.float32),
            pltpu.VMEM((_BPW,), jnp.float32),
        ],
        compiler_params=pltpu.CompilerParams(needs_layout_passes=False),
    )
    def gk(px_hbm, py_hbm, pz_hbm, idx_hbm, outx_hbm, outy_hbm, outz_hbm,
           px, py, pz, iv, ox, oy, oz):
        wid = lax.axis_index("s") * 2 + lax.axis_index("c")
        base = wid * _BPW
        pltpu.sync_copy(px_hbm, px)
        pltpu.sync_copy(py_hbm, py)
        pltpu.sync_copy(pz_hbm, pz)
        pltpu.sync_copy(idx_hbm.at[pl.ds(base, _BPW)], iv)

        def body(i, _):
            sl = pl.ds(i * 16, 16)
            ids = iv[sl]
            ox[sl] = plsc.load_gather(px, [ids])
            oy[sl] = plsc.load_gather(py, [ids])
            oz[sl] = plsc.load_gather(pz, [ids])
            return 0

        lax.fori_loop(0, _BPW // 16, body, 0)
        pltpu.sync_copy(ox, outx_hbm.at[pl.ds(base, _BPW)])
        pltpu.sync_copy(oy, outy_hbm.at[pl.ds(base, _BPW)])
        pltpu.sync_copy(oz, outz_hbm.at[pl.ds(base, _BPW)])

    return gk(posx, posy, posz, idx_flat)


def _post_body(pos_ref, rot_ref, nx_ref, ny_ref, nz_ref,
               emb_ref, np_ref, nd_ref):
    pos = pos_ref[...]
    rot = rot_ref[...]
    io = lax.broadcasted_iota(jnp.int32, (1, _HALF), 1).astype(jnp.float32)
    freqs = jnp.exp(-np.log(10000.0) * io / _HALF)

    relx = nx_ref[...] - pos[:, 0:1]
    rely = ny_ref[...] - pos[:, 1:2]
    relz = nz_ref[...] - pos[:, 2:3]
    lx = rot[:, 0:1] * relx + rot[:, 3:4] * rely + rot[:, 6:7] * relz
    ly = rot[:, 1:2] * relx + rot[:, 4:5] * rely + rot[:, 7:8] * relz
    lz = rot[:, 2:3] * relx + rot[:, 5:6] * rely + rot[:, 8:9] * relz
    np_ref[...] = jnp.stack([lx, ly, lz], axis=-1)

    norm = jnp.sqrt(lx * lx + ly * ly + lz * lz)
    args = norm[:, :, None] * freqs[None, 0, :]
    nd_ref[...] = jnp.concatenate([jnp.sin(args), jnp.cos(args)], axis=-1)

    pieces = []
    for c in range(3):
        a = pos[:, c:c + 1] * freqs
        pieces.append(jnp.sin(a))
        pieces.append(jnp.cos(a))
    emb_ref[...] = jnp.concatenate(pieces, axis=1)


def _post_call(pos, rot9, nx, ny, nz):
    return pl.pallas_call(
        _post_body,
        grid=(_NBLK,),
        in_specs=[
            pl.BlockSpec((_RB, 3), lambda b: (b, 0)),
            pl.BlockSpec((_RB, 9), lambda b: (b, 0)),
            pl.BlockSpec((_RB, _K), lambda b: (b, 0)),
            pl.BlockSpec((_RB, _K), lambda b: (b, 0)),
            pl.BlockSpec((_RB, _K), lambda b: (b, 0)),
        ],
        out_specs=[
            pl.BlockSpec((_RB, 3 * _PED), lambda b: (b, 0)),
            pl.BlockSpec((_RB, _K, 3), lambda b: (b, 0, 0)),
            pl.BlockSpec((_RB, _K, _PED), lambda b: (b, 0, 0)),
        ],
        out_shape=[
            jax.ShapeDtypeStruct((_N, 3 * _PED), jnp.float32),
            jax.ShapeDtypeStruct((_N, _K, 3), jnp.float32),
            jax.ShapeDtypeStruct((_N, _K, _PED), jnp.float32),
        ],
    )(pos, rot9, nx, ny, nz)


def kernel(affines):
    n = affines.shape[0]
    positions = affines[:, :3, 3]
    rot9 = affines[:, :3, :3].reshape(n, 9)
    ppad = jnp.concatenate(
        [positions, jnp.zeros((n, 5), jnp.float32)], axis=1)

    edge_index = _topk_call(ppad)

    gx, gy, gz = _gather_sc(positions[:, 0], positions[:, 1],
                            positions[:, 2], edge_index.reshape(-1))
    nx = gx.reshape(n, _K)
    ny = gy.reshape(n, _K)
    nz = gz.reshape(n, _K)

    pos3d_emb, neighbour_positions, neighbour_distances = _post_call(
        positions, rot9, nx, ny, nz)

    full_edge_index = jnp.stack(
        [edge_index.reshape(-1), jnp.repeat(jnp.arange(n), _K)], axis=0)
    return (pos3d_emb, positions, neighbour_positions, neighbour_distances,
            edge_index, full_edge_index)


# speedup vs baseline: 7.1219x; 1.4534x over previous
import functools

import numpy as np
import jax
import jax.numpy as jnp
from jax import lax
from jax.experimental import pallas as pl
from jax.experimental.pallas import tpu as pltpu
from jax.experimental.pallas import tpu_sc as plsc

_N = 8192
_K = 32
_PED = 64
_HALF = _PED // 2
_RB = 128
_NBLK = _N // _RB
_CN = _N // 128
_TMAX = _CN
_PRB = 256
_PNB = _N // _PRB
_NW = 32
_BPW = _N * _K // _NW


def _topk_body(pt_ref, pb_ref, idx_ref, d_ref, pv_ref, pi_ref):
    b = pl.program_id(0)
    pt = pt_ref[...]
    pb = pb_ref[...]
    sqc = jnp.sum(pt * pt, axis=0, keepdims=True)
    sqr = jnp.sum(pb * pb, axis=1, keepdims=True)
    dot = lax.dot_general(pb, pt, (((1,), (0,)), ((), ())),
                          preferred_element_type=jnp.float32)
    d = sqr + sqc - 2.0 * dot
    row = lax.broadcasted_iota(jnp.int32, (_RB, _N), 0)
    col = lax.broadcasted_iota(jnp.int32, (_RB, _N), 1)
    d_ref[...] = jnp.where(col == row + b * _RB, jnp.inf, d)

    lane = lax.broadcasted_iota(jnp.int32, (_RB, 128), 1)

    def w_cond(state):
        t, done = state
        return jnp.logical_and(t < _TMAX, jnp.logical_not(done))

    def w_body(state):
        t, _ = state
        accv = d_ref[:, 0:128]
        acci = jnp.zeros((_RB, 128), jnp.int32)
        for c in range(1, _CN):
            v = d_ref[:, c * 128:(c + 1) * 128]
            lt = v < accv
            accv = jnp.where(lt, v, accv)
            acci = jnp.where(lt, c, acci)
        pv_ref[pl.ds(t, 1)] = accv[None]
        pi_ref[pl.ds(t, 1)] = (acci * 128 + lane)[None]
        for c in range(_CN):
            sl = pl.ds(c * 128, 128)
            d_ref[:, sl] = jnp.where(acci == c, jnp.inf, d_ref[:, sl])
        gmin = jnp.min(accv, axis=1, keepdims=True)

        def cnt_body(tt, cv):
            return cv + (pv_ref[tt] < gmin).astype(jnp.int32)

        cntv = lax.fori_loop(0, t + 1, cnt_body,
                             jnp.zeros((_RB, 128), jnp.int32))
        cnt = jnp.sum(cntv, axis=1)
        done = jnp.all(cnt >= _K)
        return t + 1, done

    t_f, _ = lax.while_loop(w_cond, w_body, (jnp.int32(0), jnp.bool_(False)))

    kio = lax.broadcasted_iota(jnp.int32, (_RB, _K), 1)
    big = jnp.float32(jnp.inf)

    def eg_body(j, out):
        def mn_body(tt, st):
            bv, bi = st
            v = pv_ref[tt]
            i = pi_ref[tt]
            better = jnp.logical_or(
                v < bv, jnp.logical_and(v == bv, i < bi))
            return jnp.where(better, v, bv), jnp.where(better, i, bi)

        bestv, besti = lax.fori_loop(
            0, t_f, mn_body,
            (jnp.full((_RB, 128), big), jnp.full((_RB, 128), _N, jnp.int32)))
        m = jnp.min(bestv, axis=1, keepdims=True)
        amin = jnp.min(jnp.where(bestv == m, besti, _N), axis=1,
                       keepdims=True)

        def msk_body(tt, _):
            pv = pv_ref[tt]
            hit = jnp.logical_and(pv == m, pi_ref[tt] == amin)
            pv_ref[pl.ds(tt, 1)] = jnp.where(hit, big, pv)[None]
            return 0

        lax.fori_loop(0, t_f, msk_body, 0)
        return jnp.where(kio == j, amin, out)

    idx_ref[...] = lax.fori_loop(0, _K, eg_body,
                                 jnp.zeros((_RB, _K), jnp.int32))


def _topk_call(ppad):
    return pl.pallas_call(
        _topk_body,
        grid=(_NBLK,),
        in_specs=[
            pl.BlockSpec((8, _N), lambda b: (0, 0)),
            pl.BlockSpec((_RB, 8), lambda b: (b, 0)),
        ],
        out_specs=pl.BlockSpec((_RB, _K), lambda b: (b, 0)),
        out_shape=jax.ShapeDtypeStruct((_N, _K), jnp.int32),
        scratch_shapes=[
            pltpu.VMEM((_RB, _N), jnp.float32),
            pltpu.VMEM((_TMAX, _RB, 128), jnp.float32),
            pltpu.VMEM((_TMAX, _RB, 128), jnp.int32),
        ],
    )(ppad.T, ppad)


def _gather_sc(posx, posy, posz, idx_flat):
    mesh = plsc.VectorSubcoreMesh(core_axis_name="c", subcore_axis_name="s")

    @functools.partial(
        pl.kernel,
        mesh=mesh,
        out_type=[
            jax.ShapeDtypeStruct((_N * _K,), jnp.float32),
            jax.ShapeDtypeStruct((_N * _K,), jnp.float32),
            jax.ShapeDtypeStruct((_N * _K,), jnp.float32),
        ],
        scratch_types=[
            pltpu.VMEM((_N,), jnp.float32),
            pltpu.VMEM((_N,), jnp.float32),
            pltpu.VMEM((_N,), jnp.float32),
            pltpu.VMEM((_BPW,), jnp.int32),
            pltpu.VMEM((_BPW,), jnp.float32),
            pltpu.VMEM((_BPW,), jnp.float32),
            pltpu.VMEM((_BPW,), jnp.float32),
        ],
        compiler_params=pltpu.CompilerParams(needs_layout_passes=False),
    )
    def gk(px_hbm, py_hbm, pz_hbm, idx_hbm, outx_hbm, outy_hbm, outz_hbm,
           px, py, pz, iv, ox, oy, oz):
        wid = lax.axis_index("s") * 2 + lax.axis_index("c")
        base = wid * _BPW
        pltpu.sync_copy(px_hbm, px)
        pltpu.sync_copy(py_hbm, py)
        pltpu.sync_copy(pz_hbm, pz)
        pltpu.sync_copy(idx_hbm.at[pl.ds(base, _BPW)], iv)

        def body(i, _):
            sl = pl.ds(i * 16, 16)
            ids = iv[sl]
            ox[sl] = plsc.load_gather(px, [ids])
            oy[sl] = plsc.load_gather(py, [ids])
            oz[sl] = plsc.load_gather(pz, [ids])
            return 0

        lax.fori_loop(0, _BPW // 16, body, 0)
        pltpu.sync_copy(ox, outx_hbm.at[pl.ds(base, _BPW)])
        pltpu.sync_copy(oy, outy_hbm.at[pl.ds(base, _BPW)])
        pltpu.sync_copy(oz, outz_hbm.at[pl.ds(base, _BPW)])

    return gk(posx, posy, posz, idx_flat)


def _post_body(pos_ref, rot_ref, nx_ref, ny_ref, nz_ref,
               emb_ref, np_ref, nd_ref):
    pos = pos_ref[...]
    rot = rot_ref[...]
    io = lax.broadcasted_iota(jnp.int32, (1, _HALF), 1).astype(jnp.float32)
    freqs = jnp.exp(-np.log(10000.0) * io / _HALF)

    relx = nx_ref[...] - pos[:, 0:1]
    rely = ny_ref[...] - pos[:, 1:2]
    relz = nz_ref[...] - pos[:, 2:3]
    lx = rot[:, 0:1] * relx + rot[:, 3:4] * rely + rot[:, 6:7] * relz
    ly = rot[:, 1:2] * relx + rot[:, 4:5] * rely + rot[:, 7:8] * relz
    lz = rot[:, 2:3] * relx + rot[:, 5:6] * rely + rot[:, 8:9] * relz
    np_ref[...] = jnp.stack([lx, ly, lz], axis=-1)

    norm = jnp.sqrt(lx * lx + ly * ly + lz * lz)
    args = norm[:, :, None] * freqs[None, 0, :]
    nd_ref[...] = jnp.concatenate([jnp.sin(args), jnp.cos(args)], axis=-1)

    pieces = []
    for c in range(3):
        a = pos[:, c:c + 1] * freqs
        pieces.append(jnp.sin(a))
        pieces.append(jnp.cos(a))
    emb_ref[...] = jnp.concatenate(pieces, axis=1)


def _post_call(pos, rot9, nx, ny, nz):
    return pl.pallas_call(
        _post_body,
        grid=(_PNB,),
        in_specs=[
            pl.BlockSpec((_PRB, 3), lambda b: (b, 0)),
            pl.BlockSpec((_PRB, 9), lambda b: (b, 0)),
            pl.BlockSpec((_PRB, _K), lambda b: (b, 0)),
            pl.BlockSpec((_PRB, _K), lambda b: (b, 0)),
            pl.BlockSpec((_PRB, _K), lambda b: (b, 0)),
        ],
        out_specs=[
            pl.BlockSpec((_PRB, 3 * _PED), lambda b: (b, 0)),
            pl.BlockSpec((_PRB, _K, 3), lambda b: (b, 0, 0)),
            pl.BlockSpec((_PRB, _K, _PED), lambda b: (b, 0, 0)),
        ],
        out_shape=[
            jax.ShapeDtypeStruct((_N, 3 * _PED), jnp.float32),
            jax.ShapeDtypeStruct((_N, _K, 3), jnp.float32),
            jax.ShapeDtypeStruct((_N, _K, _PED), jnp.float32),
        ],
    )(pos, rot9, nx, ny, nz)


def kernel(affines):
    n = affines.shape[0]
    positions = affines[:, :3, 3]
    rot9 = affines[:, :3, :3].reshape(n, 9)
    ppad = jnp.concatenate(
        [positions, jnp.zeros((n, 5), jnp.float32)], axis=1)

    edge_index = _topk_call(ppad)

    gx, gy, gz = _gather_sc(positions[:, 0], positions[:, 1],
                            positions[:, 2], edge_index.reshape(-1))
    nx = gx.reshape(n, _K)
    ny = gy.reshape(n, _K)
    nz = gz.reshape(n, _K)

    pos3d_emb, neighbour_positions, neighbour_distances = _post_call(
        positions, rot9, nx, ny, nz)

    full_edge_index = jnp.stack(
        [edge_index.reshape(-1), jnp.repeat(jnp.arange(n), _K)], axis=0)
    return (pos3d_emb, positions, neighbour_positions, neighbour_distances,
            edge_index, full_edge_index)
